# baseline (speedup 1.0000x reference)
import functools

import jax
import jax.numpy as jnp
from jax import lax
from jax.experimental import pallas as pl
from jax.experimental.pallas import tpu as pltpu
from jax.experimental.pallas import tpu_sc as plsc

B, S, D = 4, 2048, 1024
NC, NS = 2, 16
NW = NC * NS
RW = S // NW
CH = 8
NCH = RW // CH
LANES = 16
VPR = D // LANES

_mesh = plsc.VectorSubcoreMesh(core_axis_name="c", subcore_axis_name="s")


@functools.partial(
    pl.kernel,
    mesh=_mesh,
    out_type=jax.ShapeDtypeStruct((B, S, D), jnp.float32),
    scratch_types=[
        pltpu.VMEM((3, B, CH, D), jnp.float32),
        pltpu.VMEM((3, CH, D), jnp.float32),
        pltpu.SemaphoreType.DMA,
        pltpu.SemaphoreType.DMA,
        pltpu.SemaphoreType.DMA,
        pltpu.SemaphoreType.DMA,
        pltpu.SemaphoreType.DMA,
        pltpu.SemaphoreType.DMA,
        pltpu.SMEM((8,), jnp.int32),
    ],
)
def _pos_add(x_hbm, emb_hbm, out_hbm, x_buf, emb_buf,
             in_sem0, in_sem1, in_sem2, out_sem0, out_sem1, out_sem2,
             smem_scratch):
    wid = lax.axis_index("s") * NC + lax.axis_index("c")
    row0 = wid * RW

    in_sems = (in_sem0, in_sem1, in_sem2)
    out_sems = (out_sem0, out_sem1, out_sem2)

    def load(j, slot):
        rows = pl.multiple_of(row0 + j * CH, CH)
        return [pltpu.async_copy(emb_hbm.at[pl.ds(rows, CH)],
                                 emb_buf.at[slot], in_sems[slot]),
                pltpu.async_copy(x_hbm.at[:, pl.ds(rows, CH)],
                                 x_buf.at[slot], in_sems[slot])]

    def store(j, slot):
        rows = pl.multiple_of(row0 + j * CH, CH)
        return [pltpu.async_copy(x_buf.at[slot],
                                 out_hbm.at[:, pl.ds(rows, CH)],
                                 out_sems[slot])]

    def compute(slot):
        def step(i, carry):
            o = i * LANES
            vs = [emb_buf[slot, r, pl.ds(o, LANES)] for r in range(CH)]
            for r in range(CH):
                for b in range(B):
                    plsc.addupdate(x_buf.at[slot, b, r, pl.ds(o, LANES)],
                                   vs[r])
            return carry
        lax.fori_loop(0, VPR, step, 0)

    skew = lax.fori_loop(0, (wid % 2) * 1600,
                         lambda i, c: (c ^ (c << 13)) + i, 1)
    smem_scratch[0] = skew

    NSLOT = 3
    pend_out = {s: [] for s in range(NSLOT)}
    pend_in = {s: [] for s in range(NSLOT)}
    pend_in[0] = load(0, 0)
    pend_in[1] = load(1, 1)
    for j in range(NCH):
        slot = j % NSLOT
        nxt = (j + 2) % NSLOT
        if j + 2 < NCH:
            for h in pend_out[nxt]:
                h.wait()
            pend_in[nxt] = load(j + 2, nxt)
        for h in pend_in[slot]:
            h.wait()
        compute(slot)
        pend_out[slot] = store(j, slot)
    for sl in range(NSLOT):
        for h in pend_out[sl]:
            h.wait()


def kernel(x, emb):
    return _pos_add(x, emb)


# speedup vs baseline: 1.0123x; 1.0123x over previous
import functools

import jax
import jax.numpy as jnp
from jax import lax
from jax.experimental import pallas as pl
from jax.experimental.pallas import tpu as pltpu
from jax.experimental.pallas import tpu_sc as plsc

B, S, D = 4, 2048, 1024
NC, NS = 2, 16
NW = NC * NS
RW = S // NW
CH = 8
NCH = RW // CH
LANES = 16
VPR = D // LANES
NSLOT = 3

_mesh = plsc.VectorSubcoreMesh(core_axis_name="c", subcore_axis_name="s")


@functools.partial(
    pl.kernel,
    mesh=_mesh,
    out_type=jax.ShapeDtypeStruct((B, S, D), jnp.float32),
    scratch_types=[
        pltpu.VMEM((NSLOT, B, CH, D), jnp.float32),
        pltpu.VMEM((NSLOT, CH, D), jnp.float32),
        pltpu.SemaphoreType.DMA((NSLOT,)),
        pltpu.SemaphoreType.DMA((NSLOT, B)),
        pltpu.SemaphoreType.DMA((NSLOT,)),
    ],
)
def _pos_add(x_hbm, emb_hbm, out_hbm, x_buf, emb_buf,
             emb_sem, x_sem, out_sem):
    wid = lax.axis_index("s") * NC + lax.axis_index("c")
    row0 = wid * RW

    def load(j, slot):
        rows = pl.multiple_of(row0 + j * CH, CH)
        hs = [pltpu.async_copy(emb_hbm.at[pl.ds(rows, CH)],
                               emb_buf.at[slot], emb_sem.at[slot])]
        for b in range(B):
            hs.append(pltpu.async_copy(x_hbm.at[b, pl.ds(rows, CH)],
                                       x_buf.at[slot, b], x_sem.at[slot, b]))
        return hs

    def store_b(j, slot, b):
        rows = pl.multiple_of(row0 + j * CH, CH)
        return pltpu.async_copy(x_buf.at[slot, b],
                                out_hbm.at[b, pl.ds(rows, CH)],
                                out_sem.at[slot])

    def compute_b(slot, b):
        def step(i, carry):
            o = i * LANES
            vs = [emb_buf[slot, r, pl.ds(o, LANES)] for r in range(CH)]
            for r in range(CH):
                plsc.addupdate(x_buf.at[slot, b, r, pl.ds(o, LANES)], vs[r])
            return carry
        lax.fori_loop(0, VPR, step, 0)

    pend_out = {s: [] for s in range(NSLOT)}
    pend_in = {s: [] for s in range(NSLOT)}
    pend_in[0] = load(0, 0)
    pend_in[1] = load(1, 1)
    for j in range(NCH):
        slot = j % NSLOT
        nxt = (j + 2) % NSLOT
        if j + 2 < NCH:
            for h in pend_out[nxt]:
                h.wait()
            pend_in[nxt] = load(j + 2, nxt)
        pend_in[slot][0].wait()
        outs = []
        for b in range(B):
            pend_in[slot][1 + b].wait()
            compute_b(slot, b)
            outs.append(store_b(j, slot, b))
        pend_out[slot] = outs
    for sl in range(NSLOT):
        for h in pend_out[sl]:
            h.wait()


def kernel(x, emb):
    return _pos_add(x, emb)
